# baseline (speedup 1.0000x reference)
import jax
import jax.numpy as jnp
from jax.experimental import pallas as pl
from jax.experimental.pallas import tpu as pltpu

B, S, D_MODEL, D_SPACE = 4, 4096, 2048, 64
N_FQK = 512
TM = 2048


def _router_kernel(x_ref, w_ref, b_ref, emb_ref, out_ref):
    emb = emb_ref[...]
    ss = jnp.sum(emb * emb, axis=1, keepdims=True)
    emb_n = (emb * jax.lax.rsqrt(jnp.maximum(ss, 1e-24))).astype(jnp.bfloat16)
    h = jnp.dot(x_ref[0].astype(jnp.bfloat16),
                w_ref[...].astype(jnp.bfloat16),
                preferred_element_type=jnp.float32)
    h = h + b_ref[...][None, :]
    out_ref[0] = jax.lax.dot_general(
        h.astype(jnp.bfloat16), emb_n,
        (((1,), (1,)), ((), ())),
        preferred_element_type=jnp.float32)


def kernel(x, W, b, neuron_emb):
    grid = (B, S // TM)
    out = pl.pallas_call(
        _router_kernel,
        grid=grid,
        in_specs=[
            pl.BlockSpec((1, TM, D_MODEL), lambda i, j: (i, j, 0)),
            pl.BlockSpec((D_MODEL, D_SPACE), lambda i, j: (0, 0)),
            pl.BlockSpec((D_SPACE,), lambda i, j: (0,)),
            pl.BlockSpec((N_FQK, D_SPACE), lambda i, j: (0, 0)),
        ],
        out_specs=pl.BlockSpec((1, TM, N_FQK), lambda i, j: (i, j, 0)),
        out_shape=jax.ShapeDtypeStruct((B, S, N_FQK), jnp.float32),
        compiler_params=pltpu.CompilerParams(
            dimension_semantics=("parallel", "parallel")),
    )(x, W, b, neuron_emb)
    return out


# speedup vs baseline: 1.0271x; 1.0271x over previous
import jax
import jax.numpy as jnp
from jax.experimental import pallas as pl
from jax.experimental.pallas import tpu as pltpu

B, S, D_MODEL, D_SPACE = 4, 4096, 2048, 64
N_FQK = 512
TM = 2048
KH = D_MODEL // 2


def _router_kernel(xa_ref, xb_ref, wa_ref, wb_ref, b_ref, emb_ref, out_ref):
    emb = emb_ref[...]
    ss = jnp.sum(emb * emb, axis=1, keepdims=True)
    emb_n = (emb * jax.lax.rsqrt(jnp.maximum(ss, 1e-24))).astype(jnp.bfloat16)
    h = jnp.dot(xa_ref[0].astype(jnp.bfloat16),
                wa_ref[...].astype(jnp.bfloat16),
                preferred_element_type=jnp.float32)
    h = h + jnp.dot(xb_ref[0].astype(jnp.bfloat16),
                    wb_ref[...].astype(jnp.bfloat16),
                    preferred_element_type=jnp.float32)
    h = h + b_ref[...]
    out_ref[0] = jax.lax.dot_general(
        h.astype(jnp.bfloat16), emb_n,
        (((1,), (1,)), ((), ())),
        preferred_element_type=jnp.float32)


def kernel(x, W, b, neuron_emb):
    emb = neuron_emb[:N_FQK]
    b2 = b.reshape(1, D_SPACE)
    grid = (B, S // TM)
    out = pl.pallas_call(
        _router_kernel,
        grid=grid,
        in_specs=[
            pl.BlockSpec((1, TM, KH), lambda i, j: (i, j, 0)),
            pl.BlockSpec((1, TM, KH), lambda i, j: (i, j, 1)),
            pl.BlockSpec((KH, D_SPACE), lambda i, j: (0, 0)),
            pl.BlockSpec((KH, D_SPACE), lambda i, j: (1, 0)),
            pl.BlockSpec((1, D_SPACE), lambda i, j: (0, 0)),
            pl.BlockSpec((N_FQK, D_SPACE), lambda i, j: (0, 0)),
        ],
        out_specs=pl.BlockSpec((1, TM, N_FQK), lambda i, j: (i, j, 0)),
        out_shape=jax.ShapeDtypeStruct((B, S, N_FQK), jnp.float32),
        compiler_params=pltpu.CompilerParams(
            dimension_semantics=("parallel", "parallel")),
    )(x, x, W, W, b2, emb)
    return out


# speedup vs baseline: 1.0304x; 1.0032x over previous
import jax
import jax.numpy as jnp
from jax.experimental import pallas as pl
from jax.experimental.pallas import tpu as pltpu

B, S, D_MODEL, D_SPACE = 4, 4096, 2048, 64
N_FQK = 512
T = B * S
CH = 1024
NCH = T // CH
NBUF = 3


def _router_kernel(x_hbm, w_ref, b_ref, emb_ref, out_hbm,
                   xbuf, obuf, isem, osem):
    wb = w_ref[...].astype(jnp.bfloat16)
    bias = b_ref[...]
    emb = emb_ref[...]
    ss = jnp.sum(emb * emb, axis=1, keepdims=True)
    emb_n = (emb * jax.lax.rsqrt(jnp.maximum(ss, 1e-24))).astype(jnp.bfloat16)

    def in_copy(i, slot):
        return pltpu.make_async_copy(
            x_hbm.at[pl.ds(i * CH, CH), :], xbuf.at[slot], isem.at[slot])

    def out_copy(i, slot):
        return pltpu.make_async_copy(
            obuf.at[slot], out_hbm.at[pl.ds(i * CH, CH), :], osem.at[slot])

    for k in range(NBUF - 1):
        in_copy(k, k).start()

    def body(i, carry):
        slot = jax.lax.rem(i, NBUF)
        nxt = i + NBUF - 1

        @pl.when(nxt < NCH)
        def _():
            in_copy(nxt, jax.lax.rem(nxt, NBUF)).start()

        in_copy(i, slot).wait()
        h = jnp.dot(xbuf[slot].astype(jnp.bfloat16), wb,
                    preferred_element_type=jnp.float32)
        h = h + bias
        lg = jax.lax.dot_general(
            h.astype(jnp.bfloat16), emb_n,
            (((1,), (1,)), ((), ())),
            preferred_element_type=jnp.float32)

        @pl.when(i >= NBUF)
        def _():
            out_copy(i - NBUF, slot).wait()

        obuf[slot] = lg
        out_copy(i, slot).start()
        return carry

    jax.lax.fori_loop(0, NCH, body, 0, unroll=False)

    for k in range(NBUF):
        i = NCH - NBUF + k
        out_copy(jnp.int32(i), jax.lax.rem(jnp.int32(i), NBUF)).wait()


def kernel(x, W, b, neuron_emb):
    x2 = x.reshape(T, D_MODEL)
    emb = neuron_emb[:N_FQK]
    b2 = b.reshape(1, D_SPACE)
    out = pl.pallas_call(
        _router_kernel,
        in_specs=[
            pl.BlockSpec(memory_space=pl.ANY),
            pl.BlockSpec((D_MODEL, D_SPACE), lambda: (0, 0)),
            pl.BlockSpec((1, D_SPACE), lambda: (0, 0)),
            pl.BlockSpec((N_FQK, D_SPACE), lambda: (0, 0)),
        ],
        out_specs=pl.BlockSpec(memory_space=pl.ANY),
        out_shape=jax.ShapeDtypeStruct((T, N_FQK), jnp.float32),
        scratch_shapes=[
            pltpu.VMEM((NBUF, CH, D_MODEL), jnp.float32),
            pltpu.VMEM((NBUF, CH, N_FQK), jnp.float32),
            pltpu.SemaphoreType.DMA((NBUF,)),
            pltpu.SemaphoreType.DMA((NBUF,)),
        ],
    )(x2, W, b2, emb)
    return out.reshape(B, S, N_FQK)


# speedup vs baseline: 1.0483x; 1.0174x over previous
import jax
import jax.numpy as jnp
from jax.experimental import pallas as pl
from jax.experimental.pallas import tpu as pltpu

B, S, D_MODEL, D_SPACE = 4, 4096, 2048, 64
N_FQK = 512
T = B * S
CH = 512
NCH = T // CH
NBUF = 6


def _router_kernel(x_hbm, w_ref, b_ref, emb_ref, out_hbm,
                   xbuf, obuf, isem, osem):
    wb = w_ref[...].astype(jnp.bfloat16)
    bias = b_ref[...]
    emb = emb_ref[...]
    ss = jnp.sum(emb * emb, axis=1, keepdims=True)
    emb_n = (emb * jax.lax.rsqrt(jnp.maximum(ss, 1e-24))).astype(jnp.bfloat16)

    def in_copy(i, slot):
        return pltpu.make_async_copy(
            x_hbm.at[pl.ds(i * CH, CH), :], xbuf.at[slot], isem.at[slot])

    def out_copy(i, slot):
        return pltpu.make_async_copy(
            obuf.at[slot], out_hbm.at[pl.ds(i * CH, CH), :], osem.at[slot])

    for k in range(NBUF - 1):
        in_copy(k, k).start()

    def body(i, carry):
        slot = jax.lax.rem(i, NBUF)
        nxt = i + NBUF - 1

        @pl.when(nxt < NCH)
        def _():
            in_copy(nxt, jax.lax.rem(nxt, NBUF)).start()

        in_copy(i, slot).wait()
        h = jnp.dot(xbuf[slot].astype(jnp.bfloat16), wb,
                    preferred_element_type=jnp.float32)
        h = h + bias
        lg = jax.lax.dot_general(
            h.astype(jnp.bfloat16), emb_n,
            (((1,), (1,)), ((), ())),
            preferred_element_type=jnp.float32)

        @pl.when(i >= NBUF)
        def _():
            out_copy(i - NBUF, slot).wait()

        obuf[slot] = lg
        out_copy(i, slot).start()
        return carry

    jax.lax.fori_loop(0, NCH, body, 0, unroll=False)

    for k in range(NBUF):
        i = NCH - NBUF + k
        out_copy(jnp.int32(i), jax.lax.rem(jnp.int32(i), NBUF)).wait()


def kernel(x, W, b, neuron_emb):
    x2 = x.reshape(T, D_MODEL)
    emb = neuron_emb[:N_FQK]
    b2 = b.reshape(1, D_SPACE)
    out = pl.pallas_call(
        _router_kernel,
        in_specs=[
            pl.BlockSpec(memory_space=pl.ANY),
            pl.BlockSpec((D_MODEL, D_SPACE), lambda: (0, 0)),
            pl.BlockSpec((1, D_SPACE), lambda: (0, 0)),
            pl.BlockSpec((N_FQK, D_SPACE), lambda: (0, 0)),
        ],
        out_specs=pl.BlockSpec(memory_space=pl.ANY),
        out_shape=jax.ShapeDtypeStruct((T, N_FQK), jnp.float32),
        scratch_shapes=[
            pltpu.VMEM((NBUF, CH, D_MODEL), jnp.float32),
            pltpu.VMEM((NBUF, CH, N_FQK), jnp.float32),
            pltpu.SemaphoreType.DMA((NBUF,)),
            pltpu.SemaphoreType.DMA((NBUF,)),
        ],
    )(x2, W, b2, emb)
    return out.reshape(B, S, N_FQK)
